# Initial kernel scaffold; baseline (speedup 1.0000x reference)
import jax
import jax.numpy as jnp
from jax.experimental import pallas as pl


def kernel(data, query, channel_index):
    raise NotImplementedError("write your pallas kernel here")


# speedup vs baseline: 2.3860x; 2.3860x over previous
import jax
import jax.numpy as jnp
from jax import lax
from jax.experimental import pallas as pl
from jax.experimental.pallas import tpu as pltpu


def _body(query_ref, chan_ref, data_ref, out_ref):
    n = data_ref.shape[1]
    nq = query_ref.shape[1]
    cols = lax.broadcasted_iota(jnp.int32, (1, n), 1)
    m = jnp.zeros((1, n), jnp.bool_)
    for k in range(nq):
        m = m | (cols == query_ref[0, k])
    out_ref[...] = jnp.where(m, chan_ref[...], data_ref[...])


def kernel(data, query, channel_index):
    m, n = data.shape
    nq = query.shape[0]
    q2 = query.astype(jnp.int32).reshape(1, nq)
    chan = channel_index.astype(data.dtype).reshape(m, 1)
    br = 512
    grid = (m // br,)
    return pl.pallas_call(
        _body,
        grid=grid,
        in_specs=[
            pl.BlockSpec((1, nq), lambda i: (0, 0), memory_space=pltpu.SMEM),
            pl.BlockSpec((br, 1), lambda i: (i, 0)),
            pl.BlockSpec((br, n), lambda i: (i, 0)),
        ],
        out_specs=pl.BlockSpec((br, n), lambda i: (i, 0)),
        out_shape=jax.ShapeDtypeStruct((m, n), data.dtype),
        compiler_params=pltpu.CompilerParams(
            dimension_semantics=("arbitrary",),
        ),
    )(q2, chan, data)
